# baseline (speedup 1.0000x reference)
import jax
import jax.numpy as jnp
from jax import lax
from jax.experimental import pallas as pl
from jax.experimental.pallas import tpu as pltpu
from jax.experimental.pallas import tpu_sc as plsc

NC = 2
NS = 16
NW = NC * NS
L = 16

B = 16384
D = 32
BPW = B // NW
NCHUNK = 4
CHUNK = BPW // NCHUNK

NROWS = 1000000
NGRAN = D * NROWS // 16


def _body(uid_hbm, mid_hbm, ut_hbm, mt_hbm, w_hbm, b_hbm, out_hbm,
          uidx, midx, ulane, mlane,
          uoffb0, uoffb1, moffb0, moffb1,
          ubuf0, ubuf1, mbuf0, mbuf1,
          outv, wv, bv, usem0, usem1, msem0, msem1):
    wid = lax.axis_index("s") * NC + lax.axis_index("c")


    pltpu.sync_copy(uid_hbm.at[pl.ds(wid * NCHUNK, NCHUNK)], uidx)
    pltpu.sync_copy(mid_hbm.at[pl.ds(wid * NCHUNK, NCHUNK)], midx)
    pltpu.sync_copy(w_hbm, wv)
    pltpu.sync_copy(b_hbm, bv)

    for j in range(NCHUNK):
        for t in range(CHUNK // L):
            sl = (j, pl.ds(t * L, L))
            fl = pl.ds(j * CHUNK + t * L, L)
            i = uidx[sl]
            ulane[fl] = jnp.bitwise_and(i, 15)
            uidx[sl] = lax.shift_right_logical(i, 4)
            i = midx[sl]
            mlane[fl] = jnp.bitwise_and(i, 15)
            midx[sl] = lax.shift_right_logical(i, 4)

    bvec = bv[pl.ds(0, L)]
    for g in range(BPW // L):
        outv[pl.ds(g * L, L)] = bvec

    iota = lax.iota(jnp.int32, L)

    def prep_issue(d, offb_u, offb_m, bu, bm, su, sm):
        base = d * (NROWS // 16)
        for j in range(NCHUNK):
            for t in range(CHUNK // L):
                sl = (j, pl.ds(t * L, L))
                offb_u[sl] = uidx[sl] + base
                offb_m[sl] = midx[sl] + base
        for j in range(NCHUNK):
            pltpu.async_copy(
                ut_hbm.at[offb_u.at[j]], bu.at[pl.ds(j * CHUNK, CHUNK)], su)
            pltpu.async_copy(
                mt_hbm.at[offb_m.at[j]], bm.at[pl.ds(j * CHUNK, CHUNK)], sm)

    def wait_bufs(bu, bm, su, sm):
        pltpu.make_async_copy(ut_hbm.at[pl.ds(0, BPW)], bu, su).wait()
        pltpu.make_async_copy(mt_hbm.at[pl.ds(0, BPW)], bm, sm).wait()

    def compute_d(d, bu, bm):
        wd = wv[pl.ds(d * L, L)]

        @pl.loop(0, BPW // L)
        def _(g):
            rows = g * L + iota
            cols = pl.ds(g * L, L)
            gu = plsc.load_gather(bu, [rows, ulane[cols]])
            gm = plsc.load_gather(bm, [rows, mlane[cols]])
            outv[cols] = outv[cols] + gu * gm * wd

    d0 = jnp.int32(0)
    prep_issue(d0, uoffb0, moffb0, ubuf0, mbuf0, usem0, msem0)
    prep_issue(d0 + 1, uoffb1, moffb1, ubuf1, mbuf1, usem1, msem1)

    @pl.loop(0, D // 2)
    def _(k):
        da = k * 2
        wait_bufs(ubuf0, mbuf0, usem0, msem0)
        compute_d(da, ubuf0, mbuf0)

        @pl.when(da + 2 < D)
        def _():
            prep_issue(da + 2, uoffb0, moffb0, ubuf0, mbuf0, usem0, msem0)

        db = da + 1
        wait_bufs(ubuf1, mbuf1, usem1, msem1)
        compute_d(db, ubuf1, mbuf1)

        @pl.when(db + 2 < D)
        def _():
            prep_issue(db + 2, uoffb1, moffb1, ubuf1, mbuf1, usem1, msem1)

    pltpu.sync_copy(outv, out_hbm.at[pl.ds(wid * BPW, BPW)])


def kernel(user_id, movie_id, user_table, movie_table, fc_w, fc_b):
    uid = user_id.astype(jnp.int32).reshape(NW * NCHUNK, CHUNK)
    mid = movie_id.astype(jnp.int32).reshape(NW * NCHUNK, CHUNK)
    ut_g = user_table.T.reshape(NGRAN, 16)
    mt_g = movie_table.T.reshape(NGRAN, 16)
    wsp = jnp.broadcast_to(fc_w.reshape(D, 1), (D, L)).reshape(D * L)
    b128 = jnp.broadcast_to(fc_b, (128,))

    cp = pltpu.CompilerParams(
        needs_layout_passes=False, use_tc_tiling_on_sc=False)

    run = pl.kernel(
        _body,
        out_type=jax.ShapeDtypeStruct((B,), jnp.float32),
        mesh=plsc.VectorSubcoreMesh(core_axis_name="c", subcore_axis_name="s"),
        compiler_params=cp,
        scratch_types=[
            pltpu.VMEM((NCHUNK, CHUNK), jnp.int32),
            pltpu.VMEM((NCHUNK, CHUNK), jnp.int32),
            pltpu.VMEM((BPW,), jnp.int32),
            pltpu.VMEM((BPW,), jnp.int32),
            pltpu.VMEM((NCHUNK, CHUNK), jnp.int32),
            pltpu.VMEM((NCHUNK, CHUNK), jnp.int32),
            pltpu.VMEM((NCHUNK, CHUNK), jnp.int32),
            pltpu.VMEM((NCHUNK, CHUNK), jnp.int32),
            pltpu.VMEM((BPW, 16), jnp.float32),
            pltpu.VMEM((BPW, 16), jnp.float32),
            pltpu.VMEM((BPW, 16), jnp.float32),
            pltpu.VMEM((BPW, 16), jnp.float32),
            pltpu.VMEM((BPW,), jnp.float32),
            pltpu.VMEM((D * L,), jnp.float32),
            pltpu.VMEM((128,), jnp.float32),
            pltpu.SemaphoreType.DMA,
            pltpu.SemaphoreType.DMA,
            pltpu.SemaphoreType.DMA,
            pltpu.SemaphoreType.DMA,
        ],
    )
    return run(uid, mid, ut_g, mt_g, wsp, b128)


# speedup vs baseline: 13.6329x; 13.6329x over previous
import jax
import jax.numpy as jnp
from jax import lax
from jax.experimental import pallas as pl
from jax.experimental.pallas import tpu as pltpu
from jax.experimental.pallas import tpu_sc as plsc

NC = 2
NS = 16
NW = NC * NS
L = 16

B = 16384
D = 32
BPW = B // NW
NCHUNK = 4
CHUNK = BPW // NCHUNK

NROWS = 1000000
MAIN = 999936
TAIL = NROWS - MAIN
ROWW = NROWS + TAIL
GPD = ROWW // 16
NGRAN = D * GPD

DW = 1536
NCHT = MAIN // DW
CPW = -(-NCHT // NW)


def _detile_body(ut_hbm, mt_hbm, tu_hbm, tm_hbm, ou_hbm, om_hbm,
                 b0, b1, tailb, so0, so1):
    wid = lax.axis_index("s") * NC + lax.axis_index("c")
    start = jnp.minimum(wid * CPW, NCHT - CPW)

    @pl.when(wid == 0)
    def _():
        for src, dst in ((tu_hbm, ou_hbm), (tm_hbm, om_hbm)):
            pltpu.sync_copy(src, tailb)
            for d in range(D):
                pltpu.sync_copy(
                    tailb.at[d], dst.at[pl.ds(d * ROWW + NROWS, TAIL)])

    @pl.loop(0, CPW)
    def _(k):
        c0 = (start + k) * DW
        n = 0
        for src, dst in ((ut_hbm, ou_hbm), (mt_hbm, om_hbm)):
            for tr in range(4):
                buf, so = (b0, so0) if n % 2 == 0 else (b1, so1)
                g = k * 8 + n

                @pl.when(g >= 2)
                def _(buf=buf, so=so):
                    pltpu.make_async_copy(
                        buf, ut_hbm.at[pl.ds(0, 8), pl.ds(0, DW)], so).wait()

                pltpu.sync_copy(src.at[pl.ds(tr * 8, 8), pl.ds(c0, DW)], buf)
                for sl in range(8):
                    pltpu.async_copy(
                        buf.at[sl],
                        dst.at[pl.ds((tr * 8 + sl) * ROWW + c0, DW)], so)
                n += 1

    for so in (so0, so1):
        pltpu.make_async_copy(
            b0, ut_hbm.at[pl.ds(0, 8), pl.ds(0, DW)], so).wait()


def _gather_body(uid_hbm, mid_hbm, ut_hbm, mt_hbm, w_hbm, b_hbm, out_hbm,
                 uidx, midx, ulane, mlane,
                 uoffb0, uoffb1, moffb0, moffb1,
                 ubuf0, ubuf1, mbuf0, mbuf1,
                 outv, wv, bv, usem0, usem1, msem0, msem1):
    wid = lax.axis_index("s") * NC + lax.axis_index("c")

    pltpu.sync_copy(uid_hbm.at[pl.ds(wid * NCHUNK, NCHUNK)], uidx)
    pltpu.sync_copy(mid_hbm.at[pl.ds(wid * NCHUNK, NCHUNK)], midx)
    pltpu.sync_copy(w_hbm, wv)
    pltpu.sync_copy(b_hbm, bv)

    for j in range(NCHUNK):
        for t in range(CHUNK // L):
            sl = (j, pl.ds(t * L, L))
            fl = pl.ds(j * CHUNK + t * L, L)
            i = uidx[sl]
            ulane[fl] = jnp.bitwise_and(i, 15)
            uidx[sl] = lax.shift_right_logical(i, 4) + lax.shift_left(
                (i >= MAIN).astype(jnp.int32), 2)
            i = midx[sl]
            mlane[fl] = jnp.bitwise_and(i, 15)
            midx[sl] = lax.shift_right_logical(i, 4) + lax.shift_left(
                (i >= MAIN).astype(jnp.int32), 2)

    bvec = bv[pl.ds(0, L)]
    for g in range(BPW // L):
        outv[pl.ds(g * L, L)] = bvec

    iota = lax.iota(jnp.int32, L)

    def prep_issue(d, offb_u, offb_m, bu, bm, su, sm):
        base = d * GPD
        for j in range(NCHUNK):
            for t in range(CHUNK // L):
                sl = (j, pl.ds(t * L, L))
                offb_u[sl] = uidx[sl] + base
                offb_m[sl] = midx[sl] + base
        for j in range(NCHUNK):
            pltpu.async_copy(
                ut_hbm.at[offb_u.at[j]], bu.at[pl.ds(j * CHUNK, CHUNK)], su)
            pltpu.async_copy(
                mt_hbm.at[offb_m.at[j]], bm.at[pl.ds(j * CHUNK, CHUNK)], sm)

    def wait_bufs(bu, bm, su, sm):
        pltpu.make_async_copy(ut_hbm.at[pl.ds(0, BPW)], bu, su).wait()
        pltpu.make_async_copy(mt_hbm.at[pl.ds(0, BPW)], bm, sm).wait()

    def compute_d(d, bu, bm):
        wd = wv[pl.ds(d * L, L)]

        @pl.loop(0, BPW // L)
        def _(g):
            rows = g * L + iota
            cols = pl.ds(g * L, L)
            gu = plsc.load_gather(bu, [rows, ulane[cols]])
            gm = plsc.load_gather(bm, [rows, mlane[cols]])
            outv[cols] = outv[cols] + gu * gm * wd

    d0 = jnp.int32(0)
    prep_issue(d0, uoffb0, moffb0, ubuf0, mbuf0, usem0, msem0)
    prep_issue(d0 + 1, uoffb1, moffb1, ubuf1, mbuf1, usem1, msem1)

    @pl.loop(0, D // 2)
    def _(k):
        da = k * 2
        wait_bufs(ubuf0, mbuf0, usem0, msem0)
        compute_d(da, ubuf0, mbuf0)

        @pl.when(da + 2 < D)
        def _():
            prep_issue(da + 2, uoffb0, moffb0, ubuf0, mbuf0, usem0, msem0)

        db = da + 1
        wait_bufs(ubuf1, mbuf1, usem1, msem1)
        compute_d(db, ubuf1, mbuf1)

        @pl.when(db + 2 < D)
        def _():
            prep_issue(db + 2, uoffb1, moffb1, ubuf1, mbuf1, usem1, msem1)

    pltpu.sync_copy(outv, out_hbm.at[pl.ds(wid * BPW, BPW)])


def kernel(user_id, movie_id, user_table, movie_table, fc_w, fc_b):
    uid = user_id.astype(jnp.int32).reshape(NW * NCHUNK, CHUNK)
    mid = movie_id.astype(jnp.int32).reshape(NW * NCHUNK, CHUNK)
    ut_t = user_table.T
    mt_t = movie_table.T
    tailu = user_table[MAIN:, :].T
    tailm = movie_table[MAIN:, :].T
    wsp = jnp.broadcast_to(fc_w.reshape(D, 1), (D, L)).reshape(D * L)
    b128 = jnp.broadcast_to(fc_b, (128,))

    detile = pl.kernel(
        _detile_body,
        out_type=[
            jax.ShapeDtypeStruct((D * ROWW,), jnp.float32),
            jax.ShapeDtypeStruct((D * ROWW,), jnp.float32),
        ],
        mesh=plsc.VectorSubcoreMesh(core_axis_name="c", subcore_axis_name="s"),
        compiler_params=pltpu.CompilerParams(use_tc_tiling_on_sc=True),
        scratch_types=[
            pltpu.VMEM((8, DW), jnp.float32),
            pltpu.VMEM((8, DW), jnp.float32),
            pltpu.VMEM((D, TAIL), jnp.float32),
            pltpu.SemaphoreType.DMA,
            pltpu.SemaphoreType.DMA,
        ],
    )
    ulin, mlin = detile(ut_t, mt_t, tailu, tailm)

    gather = pl.kernel(
        _gather_body,
        out_type=jax.ShapeDtypeStruct((B,), jnp.float32),
        mesh=plsc.VectorSubcoreMesh(core_axis_name="c", subcore_axis_name="s"),
        compiler_params=pltpu.CompilerParams(
            needs_layout_passes=False, use_tc_tiling_on_sc=False),
        scratch_types=[
            pltpu.VMEM((NCHUNK, CHUNK), jnp.int32),
            pltpu.VMEM((NCHUNK, CHUNK), jnp.int32),
            pltpu.VMEM((BPW,), jnp.int32),
            pltpu.VMEM((BPW,), jnp.int32),
            pltpu.VMEM((NCHUNK, CHUNK), jnp.int32),
            pltpu.VMEM((NCHUNK, CHUNK), jnp.int32),
            pltpu.VMEM((NCHUNK, CHUNK), jnp.int32),
            pltpu.VMEM((NCHUNK, CHUNK), jnp.int32),
            pltpu.VMEM((BPW, 16), jnp.float32),
            pltpu.VMEM((BPW, 16), jnp.float32),
            pltpu.VMEM((BPW, 16), jnp.float32),
            pltpu.VMEM((BPW, 16), jnp.float32),
            pltpu.VMEM((BPW,), jnp.float32),
            pltpu.VMEM((D * L,), jnp.float32),
            pltpu.VMEM((128,), jnp.float32),
            pltpu.SemaphoreType.DMA,
            pltpu.SemaphoreType.DMA,
            pltpu.SemaphoreType.DMA,
            pltpu.SemaphoreType.DMA,
        ],
    )
    return gather(uid, mid, ulin.reshape(NGRAN, 16), mlin.reshape(NGRAN, 16),
                  wsp, b128)


# speedup vs baseline: 20.4384x; 1.4992x over previous
import jax
import jax.numpy as jnp
from jax import lax
from jax.experimental import pallas as pl
from jax.experimental.pallas import tpu as pltpu
from jax.experimental.pallas import tpu_sc as plsc

NC = 2
NS = 16
NW = NC * NS
L = 16

B = 16384
D = 32
BPW = B // NW
NCHUNK = 4
CHUNK = BPW // NCHUNK

NROWS = 1000000
MAIN = 999936
TAIL = NROWS - MAIN
ROWW = NROWS + TAIL
GPD = ROWW // 16
NGRAN = D * GPD

DW = 3584
NCHT = MAIN // DW
CPW = -(-NCHT // NW)


def _detile_body(ut_hbm, mt_hbm, tu_hbm, tm_hbm, ou_hbm, om_hbm,
                 b0, b1, tailb, si0, si1, so0, so1):
    wid = lax.axis_index("s") * NC + lax.axis_index("c")
    start = jnp.minimum(wid * CPW, NCHT - CPW)

    @pl.when(wid == 0)
    def _():
        for src, dst in ((tu_hbm, ou_hbm), (tm_hbm, om_hbm)):
            pltpu.sync_copy(src, tailb)
            for d in range(D):
                pltpu.sync_copy(
                    tailb.at[d], dst.at[pl.ds(d * ROWW + NROWS, TAIL)])

    bufs = (b0, b1)
    sis = (si0, si1)
    sos = (so0, so1)
    TBL = ((ut_hbm, ou_hbm), (mt_hbm, om_hbm))

    def fire_in(src, c0, g):
        pltpu.async_copy(
            src.at[pl.ds((g % 4) * 8, 8), pl.ds(c0, DW)],
            bufs[g % 2], sis[g % 2])

    fire_in(ut_hbm, start * DW, 0)

    @pl.loop(0, CPW)
    def _(k):
        c0 = (start + k) * DW
        for n in range(8):
            tr = n % 4
            src, dst = TBL[n // 4]
            g = k * 8 + n
            p = n % 2
            np_ = (n + 1) % 2
            nsrc = TBL[((n + 1) % 8) // 4][0]

            def prefetch(nc0, ng, nsrc=nsrc, np_=np_):
                @pl.when(ng >= 2)
                def _():
                    pltpu.make_async_copy(
                        bufs[np_], ut_hbm.at[pl.ds(0, 8), pl.ds(0, DW)],
                        sos[np_]).wait()
                fire_in(nsrc, nc0, n + 1)

            if n < 7:
                prefetch(c0, g + 1)
            else:
                @pl.when(k + 1 < CPW)
                def _():
                    prefetch(c0 + DW, g + 1)

            pltpu.make_async_copy(
                src.at[pl.ds(tr * 8, 8), pl.ds(c0, DW)], bufs[p],
                sis[p]).wait()
            for sl in range(8):
                pltpu.async_copy(
                    bufs[p].at[sl],
                    dst.at[pl.ds((tr * 8 + sl) * ROWW + c0, DW)], sos[p])

    for so in (so0, so1):
        pltpu.make_async_copy(
            b0, ut_hbm.at[pl.ds(0, 8), pl.ds(0, DW)], so).wait()


def _gather_body(uid_hbm, mid_hbm, ut_hbm, mt_hbm, w_hbm, b_hbm, out_hbm,
                 uidx, midx, ulane, mlane,
                 uoffb0, uoffb1, moffb0, moffb1,
                 ubuf0, ubuf1, mbuf0, mbuf1,
                 outv, wv, bv, usem0, usem1, msem0, msem1):
    wid = lax.axis_index("s") * NC + lax.axis_index("c")

    pltpu.sync_copy(uid_hbm.at[pl.ds(wid * NCHUNK, NCHUNK)], uidx)
    pltpu.sync_copy(mid_hbm.at[pl.ds(wid * NCHUNK, NCHUNK)], midx)
    pltpu.sync_copy(w_hbm, wv)
    pltpu.sync_copy(b_hbm, bv)

    for j in range(NCHUNK):
        for t in range(CHUNK // L):
            sl = (j, pl.ds(t * L, L))
            fl = pl.ds(j * CHUNK + t * L, L)
            i = uidx[sl]
            ulane[fl] = jnp.bitwise_and(i, 15)
            uidx[sl] = lax.shift_right_logical(i, 4) + lax.shift_left(
                (i >= MAIN).astype(jnp.int32), 2)
            i = midx[sl]
            mlane[fl] = jnp.bitwise_and(i, 15)
            midx[sl] = lax.shift_right_logical(i, 4) + lax.shift_left(
                (i >= MAIN).astype(jnp.int32), 2)

    bvec = bv[pl.ds(0, L)]
    for g in range(BPW // L):
        outv[pl.ds(g * L, L)] = bvec

    iota = lax.iota(jnp.int32, L)

    def prep_issue(d, offb_u, offb_m, bu, bm, su, sm):
        base = d * GPD
        for j in range(NCHUNK):
            for t in range(CHUNK // L):
                sl = (j, pl.ds(t * L, L))
                offb_u[sl] = uidx[sl] + base
                offb_m[sl] = midx[sl] + base
        for j in range(NCHUNK):
            pltpu.async_copy(
                ut_hbm.at[offb_u.at[j]], bu.at[pl.ds(j * CHUNK, CHUNK)], su)
            pltpu.async_copy(
                mt_hbm.at[offb_m.at[j]], bm.at[pl.ds(j * CHUNK, CHUNK)], sm)

    def wait_bufs(bu, bm, su, sm):
        pltpu.make_async_copy(ut_hbm.at[pl.ds(0, BPW)], bu, su).wait()
        pltpu.make_async_copy(mt_hbm.at[pl.ds(0, BPW)], bm, sm).wait()

    def compute_d(d, bu, bm):
        wd = wv[pl.ds(d * L, L)]

        @pl.loop(0, BPW // L)
        def _(g):
            rows = g * L + iota
            cols = pl.ds(g * L, L)
            gu = plsc.load_gather(bu, [rows, ulane[cols]])
            gm = plsc.load_gather(bm, [rows, mlane[cols]])
            outv[cols] = outv[cols] + gu * gm * wd

    d0 = jnp.int32(0)
    prep_issue(d0, uoffb0, moffb0, ubuf0, mbuf0, usem0, msem0)
    prep_issue(d0 + 1, uoffb1, moffb1, ubuf1, mbuf1, usem1, msem1)

    @pl.loop(0, D // 2)
    def _(k):
        da = k * 2
        wait_bufs(ubuf0, mbuf0, usem0, msem0)
        compute_d(da, ubuf0, mbuf0)

        @pl.when(da + 2 < D)
        def _():
            prep_issue(da + 2, uoffb0, moffb0, ubuf0, mbuf0, usem0, msem0)

        db = da + 1
        wait_bufs(ubuf1, mbuf1, usem1, msem1)
        compute_d(db, ubuf1, mbuf1)

        @pl.when(db + 2 < D)
        def _():
            prep_issue(db + 2, uoffb1, moffb1, ubuf1, mbuf1, usem1, msem1)

    pltpu.sync_copy(outv, out_hbm.at[pl.ds(wid * BPW, BPW)])


def kernel(user_id, movie_id, user_table, movie_table, fc_w, fc_b):
    uid = user_id.astype(jnp.int32).reshape(NW * NCHUNK, CHUNK)
    mid = movie_id.astype(jnp.int32).reshape(NW * NCHUNK, CHUNK)
    ut_t = user_table.T
    mt_t = movie_table.T
    tailu = user_table[MAIN:, :].T
    tailm = movie_table[MAIN:, :].T
    wsp = jnp.broadcast_to(fc_w.reshape(D, 1), (D, L)).reshape(D * L)
    b128 = jnp.broadcast_to(fc_b, (128,))

    detile = pl.kernel(
        _detile_body,
        out_type=[
            jax.ShapeDtypeStruct((D * ROWW,), jnp.float32),
            jax.ShapeDtypeStruct((D * ROWW,), jnp.float32),
        ],
        mesh=plsc.VectorSubcoreMesh(core_axis_name="c", subcore_axis_name="s"),
        compiler_params=pltpu.CompilerParams(use_tc_tiling_on_sc=True),
        scratch_types=[
            pltpu.VMEM((8, DW), jnp.float32),
            pltpu.VMEM((8, DW), jnp.float32),
            pltpu.VMEM((D, TAIL), jnp.float32),
            pltpu.SemaphoreType.DMA,
            pltpu.SemaphoreType.DMA,
            pltpu.SemaphoreType.DMA,
            pltpu.SemaphoreType.DMA,
        ],
    )
    ulin, mlin = detile(ut_t, mt_t, tailu, tailm)

    gather = pl.kernel(
        _gather_body,
        out_type=jax.ShapeDtypeStruct((B,), jnp.float32),
        mesh=plsc.VectorSubcoreMesh(core_axis_name="c", subcore_axis_name="s"),
        compiler_params=pltpu.CompilerParams(
            needs_layout_passes=False, use_tc_tiling_on_sc=False),
        scratch_types=[
            pltpu.VMEM((NCHUNK, CHUNK), jnp.int32),
            pltpu.VMEM((NCHUNK, CHUNK), jnp.int32),
            pltpu.VMEM((BPW,), jnp.int32),
            pltpu.VMEM((BPW,), jnp.int32),
            pltpu.VMEM((NCHUNK, CHUNK), jnp.int32),
            pltpu.VMEM((NCHUNK, CHUNK), jnp.int32),
            pltpu.VMEM((NCHUNK, CHUNK), jnp.int32),
            pltpu.VMEM((NCHUNK, CHUNK), jnp.int32),
            pltpu.VMEM((BPW, 16), jnp.float32),
            pltpu.VMEM((BPW, 16), jnp.float32),
            pltpu.VMEM((BPW, 16), jnp.float32),
            pltpu.VMEM((BPW, 16), jnp.float32),
            pltpu.VMEM((BPW,), jnp.float32),
            pltpu.VMEM((D * L,), jnp.float32),
            pltpu.VMEM((128,), jnp.float32),
            pltpu.SemaphoreType.DMA,
            pltpu.SemaphoreType.DMA,
            pltpu.SemaphoreType.DMA,
            pltpu.SemaphoreType.DMA,
        ],
    )
    return gather(uid, mid, ulin.reshape(NGRAN, 16), mlin.reshape(NGRAN, 16),
                  wsp, b128)
